# baseline (speedup 1.0000x reference)
import functools

import jax
import jax.numpy as jnp
from jax import lax
from jax.experimental import pallas as pl
from jax.experimental.pallas import tpu as pltpu
from jax.experimental.pallas import tpu_sc as plsc

_RANK = 32
_DIM = 128
_HALF = 16
_PACK = _DIM // _HALF


def _sc_transpose_gather(table_t, idx_flat, n_rows, n_vocab):
    info = plsc.get_sparse_core_info()
    nc, ns = info.num_cores, info.num_subcores
    nw = nc * ns
    lk_per_tile = n_rows // ns
    g_chunk = 1600
    gq = g_chunk // _PACK
    n_gchunks = lk_per_tile // g_chunk
    t_chunk = 4000
    t_sub = 800
    n_tchunks_total = n_vocab // t_chunk
    n_titers = (n_tchunks_total + ns - 1) // ns
    t_groups = t_sub // 16
    mesh = plsc.VectorSubcoreMesh(core_axis_name="c", subcore_axis_name="s")

    @functools.partial(
        pl.kernel,
        mesh=mesh,
        compiler_params=pltpu.CompilerParams(
            use_tc_tiling_on_sc=False, needs_layout_passes=False
        ),
        out_type=(
            jax.ShapeDtypeStruct((n_vocab, _HALF), jnp.float32),
            jax.ShapeDtypeStruct((n_vocab, _HALF), jnp.float32),
            jax.ShapeDtypeStruct((n_rows // _PACK, _DIM), jnp.float32),
            jax.ShapeDtypeStruct((n_rows // _PACK, _DIM), jnp.float32),
        ),
        scratch_types=[
            pltpu.VMEM((_HALF, t_chunk), jnp.float32),
            pltpu.VMEM((t_sub, _HALF), jnp.float32),
            pltpu.VMEM((g_chunk,), jnp.int32),
            pltpu.VMEM((g_chunk, _HALF), jnp.float32),
            pltpu.SemaphoreType.DMA,
            pltpu.SemaphoreType.DMA,
            pltpu.SemaphoreType.DMA,
        ],
    )
    def fused_k(
        tab_hbm, idx_hbm, lin_a, lin_b, h4_a, h4_b,
        tin_v, tout_v, idx_v, rows_v, sem, sem2, sem3,
    ):
        c = lax.axis_index("c")
        t = lax.axis_index("s")
        iota16 = lax.iota(jnp.int32, 16)
        rs = [(iota16 + s) % 16 for s in range(_HALF)]

        def run(lin_ref, h4_ref, rank_base):
            def tchunk(k, carry):
                cid = k * ns + t

                @pl.when(cid < n_tchunks_total)
                def _():
                    pltpu.sync_copy(
                        tab_hbm.at[
                            pl.ds(rank_base, _HALF),
                            pl.ds(cid * t_chunk, t_chunk),
                        ],
                        tin_v,
                    )

                    def tsub(sub, carry1):
                        def tgroup(g, carry2):
                            colbase = sub * t_sub + g * 16 + iota16
                            vals = [
                                plsc.load_gather(tin_v, [rs[s], colbase])
                                for s in range(_HALF)
                            ]
                            for s in range(_HALF):
                                plsc.store_scatter(
                                    tout_v,
                                    [g * 16 + iota16, rs[s]],
                                    vals[s],
                                )
                            return carry2

                        lax.fori_loop(0, t_groups, tgroup, 0)
                        pltpu.sync_copy(
                            tout_v,
                            lin_ref.at[
                                pl.ds(cid * t_chunk + sub * t_sub, t_sub), :
                            ],
                        )
                        return carry1

                    lax.fori_loop(0, t_chunk // t_sub, tsub, 0)

                return carry

            lax.fori_loop(0, n_titers, tchunk, 0)

        @pl.when(c == 0)
        def _():
            run(lin_a, h4_a, 0)

        @pl.when(c == 1)
        def _():
            run(lin_b, h4_b, _HALF)

        plsc.subcore_barrier()

        def gather(lin_ref, h4_ref):
            lk_base = t * lk_per_tile

            def gchunk(k, carry):
                off = lk_base + k * g_chunk
                pltpu.sync_copy(idx_hbm.at[pl.ds(off, g_chunk)], idx_v)
                cps = [
                    pltpu.async_copy(
                        lin_ref.at[idx_v.at[pl.ds(p * gq, gq)]],
                        rows_v.at[pl.ds(p * gq, gq), :],
                        sem,
                    )
                    for p in range(_PACK)
                ]
                for cp in cps:
                    cp.wait()
                hrow = t * (lk_per_tile // _PACK) + k * gq
                wbs = [
                    pltpu.async_copy(
                        rows_v.at[pl.ds(p * gq, gq), :],
                        h4_ref.at[pl.ds(hrow, gq), pl.ds(p * _HALF, _HALF)],
                        sem2,
                    )
                    for p in range(_PACK)
                ]
                for wb in wbs:
                    wb.wait()
                return carry

            lax.fori_loop(0, n_gchunks, gchunk, 0)

        @pl.when(c == 0)
        def _():
            gather(lin_a, h4_a)

        @pl.when(c == 1)
        def _():
            gather(lin_b, h4_b)

    return fused_k(table_t, idx_flat)


def _tc_project(h4_a, h4_b, m_a, m_b, n_rows):
    q = 200
    grid = (n_rows // _PACK) // q

    def mm_k(xa_ref, xb_ref, ma_ref, mb_ref, out_ref):
        y = jnp.dot(xa_ref[...], ma_ref[...], preferred_element_type=jnp.float32)
        y = y + jnp.dot(
            xb_ref[...], mb_ref[...], preferred_element_type=jnp.float32
        )
        for p in range(_PACK):
            out_ref[pl.ds(p * q, q), :] = y[:, p * _DIM : (p + 1) * _DIM]

    return pl.pallas_call(
        mm_k,
        grid=(grid,),
        in_specs=[
            pl.BlockSpec((q, _DIM), lambda i: (i, 0)),
            pl.BlockSpec((q, _DIM), lambda i: (i, 0)),
            pl.BlockSpec((_DIM, _PACK * _DIM), lambda i: (0, 0)),
            pl.BlockSpec((_DIM, _PACK * _DIM), lambda i: (0, 0)),
        ],
        out_specs=pl.BlockSpec((_PACK * q, _DIM), lambda i: (i, 0)),
        out_shape=jax.ShapeDtypeStruct((n_rows, _DIM), jnp.float32),
    )(h4_a, h4_b, m_a, m_b)


def kernel(input, emb_table, linear_w):
    b, h = input.shape
    n_rows = b * h
    n_vocab = emb_table.shape[0]
    idx_flat = input.T.reshape(n_rows).astype(jnp.int32)
    table_t = emb_table.T
    _, _, h4_a, h4_b = _sc_transpose_gather(table_t, idx_flat, n_rows, n_vocab)
    wt = linear_w.T
    m_a = jax.scipy.linalg.block_diag(*([wt[:_HALF]] * _PACK))
    m_b = jax.scipy.linalg.block_diag(*([wt[_HALF:]] * _PACK))
    out = _tc_project(h4_a, h4_b, m_a, m_b, n_rows)
    return out.reshape(h, b, _DIM).transpose(1, 0, 2)


# speedup vs baseline: 4.5309x; 4.5309x over previous
import functools

import jax
import jax.numpy as jnp
from jax import lax
from jax.experimental import pallas as pl
from jax.experimental.pallas import tpu as pltpu
from jax.experimental.pallas import tpu_sc as plsc

_RANK = 32
_DIM = 128
_PACK = _DIM // _RANK


def _sc_gather_packed(table, idx_flat, n_rows):
    info = plsc.get_sparse_core_info()
    nc, ns = info.num_cores, info.num_subcores
    nw = nc * ns
    b_per_w = n_rows // nw
    chunk = 1600
    q = chunk // _PACK
    n_chunks = b_per_w // chunk
    mesh = plsc.VectorSubcoreMesh(core_axis_name="c", subcore_axis_name="s")

    @functools.partial(
        pl.kernel,
        mesh=mesh,
        compiler_params=pltpu.CompilerParams(use_tc_tiling_on_sc=False),
        out_type=jax.ShapeDtypeStruct((n_rows // _PACK, _DIM), jnp.float32),
        scratch_types=[
            pltpu.VMEM((chunk,), jnp.int32),
            pltpu.VMEM((chunk, _RANK), jnp.float32),
            pltpu.SemaphoreType.DMA,
            pltpu.SemaphoreType.DMA,
        ],
    )
    def gather_k(table_hbm, idx_hbm, out_hbm, idx_v, rows_v, sem, sem2):
        wid = lax.axis_index("s") * nc + lax.axis_index("c")
        base = wid * b_per_w

        def body(i, carry):
            off = base + i * chunk
            pltpu.sync_copy(idx_hbm.at[pl.ds(off, chunk)], idx_v)
            copies = []
            for p in range(_PACK):
                copies.append(
                    pltpu.async_copy(
                        table_hbm.at[idx_v.at[pl.ds(p * q, q)]],
                        rows_v.at[pl.ds(p * q, q), :],
                        sem,
                    )
                )
            for c in copies:
                c.wait()
            out_base = off // _PACK
            wbs = []
            for p in range(_PACK):
                wbs.append(
                    pltpu.async_copy(
                        rows_v.at[pl.ds(p * q, q), :],
                        out_hbm.at[pl.ds(out_base, q), pl.ds(p * _RANK, _RANK)],
                        sem2,
                    )
                )
            for c in wbs:
                c.wait()
            return carry

        lax.fori_loop(0, n_chunks, body, 0)

    return gather_k(table, idx_flat)


def _tc_project(rows4, m_blockdiag, n_rows):
    q = 400
    grid = (n_rows // _PACK) // q

    def mm_k(rows_ref, m_ref, out_ref):
        y = jnp.dot(rows_ref[...], m_ref[...], preferred_element_type=jnp.float32)
        for p in range(_PACK):
            out_ref[pl.ds(p * q, q), :] = y[:, p * _DIM : (p + 1) * _DIM]

    return pl.pallas_call(
        mm_k,
        grid=(grid,),
        in_specs=[
            pl.BlockSpec((q, _DIM), lambda i: (i, 0)),
            pl.BlockSpec((_DIM, _PACK * _DIM), lambda i: (0, 0)),
        ],
        out_specs=pl.BlockSpec((_PACK * q, _DIM), lambda i: (i, 0)),
        out_shape=jax.ShapeDtypeStruct((n_rows, _DIM), jnp.float32),
    )(rows4, m_blockdiag)


def kernel(input, emb_table, linear_w):
    b, h = input.shape
    n_rows = b * h
    idx_flat = input.T.reshape(n_rows).astype(jnp.int32)
    rows4 = _sc_gather_packed(emb_table, idx_flat, n_rows)
    wt = linear_w.T
    m_blockdiag = jax.scipy.linalg.block_diag(*([wt] * _PACK))
    out = _tc_project(rows4, m_blockdiag, n_rows)
    return out.reshape(h, b, _DIM).transpose(1, 0, 2)
